# baseline (speedup 1.0000x reference)
import jax
import jax.numpy as jnp
from jax.experimental import pallas as pl
from jax.experimental.pallas import tpu as pltpu

B, N, NFEAT, NHID, NCLASS, HID = 128, 256, 50, 512, 256, 256
NBLK = 16
BB = 8
BF = jnp.bfloat16


def _encoders_kernel(x1_ref, a1_ref, x2_ref, a2_ref,
                     w1a_ref, b1a_ref, w2a_ref, b2a_ref,
                     w1b_ref, b1b_ref, w2b_ref, b2b_ref,
                     h2a_ref, h2b_ref):
    w1a, w2a = w1a_ref[...].astype(BF), w2a_ref[...].astype(BF)
    w1b, w2b = w1b_ref[...].astype(BF), w2b_ref[...].astype(BF)
    b1a, b2a = b1a_ref[...], b2a_ref[...]
    b1b, b2b = b1b_ref[...], b2b_ref[...]

    step = pl.program_id(0)

    def enc(x_ref, a_ref, w1, b1, w2, b2, out_ref):
        adjs = [a_ref[i].astype(BF) for i in range(BB)]
        ax = jnp.concatenate(
            [jnp.dot(adjs[i], x_ref[step * BB + i].astype(BF), preferred_element_type=jnp.float32).astype(BF)
             for i in range(BB)], axis=0)
        h1 = jnp.maximum(jnp.dot(ax, w1, preferred_element_type=jnp.float32) + b1, 0.0).astype(BF)
        g = jnp.dot(h1, w2, preferred_element_type=jnp.float32).astype(BF)
        for i in range(BB):
            out_ref[i] = (jnp.dot(adjs[i], g[i * N:(i + 1) * N], preferred_element_type=jnp.float32)
                          + b2).astype(BF)

    enc(x1_ref, a1_ref, w1a, b1a, w2a, b2a, h2a_ref)
    enc(x2_ref, a2_ref, w1b, b1b, w2b, b2b, h2b_ref)


def _proj_fuse_kernel(a1_ref, pw1_ref, a2_ref, pw2_ref,
                      pba_ref, pbb_ref, fw_ref, fb_ref,
                      out_ref, acc1, acc2):
    k = pl.program_id(0)

    @pl.when(k == 0)
    def _init():
        acc1[...] = jnp.zeros_like(acc1)
        acc2[...] = jnp.zeros_like(acc2)

    a1 = jnp.transpose(a1_ref[...], (1, 0, 2))
    a2 = jnp.transpose(a2_ref[...], (1, 0, 2))
    c1 = acc1[...]
    c2 = acc2[...]
    for j in range(NBLK):
        c1 += jnp.dot(a1[j], pw1_ref[j].astype(BF), preferred_element_type=jnp.float32)
        c2 += jnp.dot(a2[j], pw2_ref[j].astype(BF), preferred_element_type=jnp.float32)
    acc1[...] = c1
    acc2[...] = c2

    @pl.when(k == pl.num_programs(0) - 1)
    def _fin():
        e1 = acc1[...] + pba_ref[...]
        e2 = acc2[...] + pbb_ref[...]
        out_ref[...] = (jnp.dot(e1, fw_ref[0:HID, :], preferred_element_type=jnp.float32)
                        + jnp.dot(e2, fw_ref[HID:2 * HID, :], preferred_element_type=jnp.float32)
                        + fb_ref[...])


def kernel(xs1, adjs1, xs2, adjs2, W1a, b1a, W2a, b2a, Pwa, Pba,
           W1b, b1b, W2b, b2b, Pwb, Pbb, Fw, Fb):
    b1a2, b2a2 = b1a.reshape(1, NHID), b2a.reshape(1, NCLASS)
    b1b2, b2b2 = b1b.reshape(1, NHID), b2b.reshape(1, NCLASS)

    full = lambda shape: pl.BlockSpec(shape, lambda b: (0,) * len(shape))
    h2a, h2b = pl.pallas_call(
        _encoders_kernel,
        grid=(B // BB,),
        in_specs=[
            full((B, N, NFEAT)),
            pl.BlockSpec((BB, N, N), lambda b: (b, 0, 0)),
            full((B, N, NFEAT)),
            pl.BlockSpec((BB, N, N), lambda b: (b, 0, 0)),
            full((NFEAT, NHID)), full((1, NHID)), full((NHID, NCLASS)), full((1, NCLASS)),
            full((NFEAT, NHID)), full((1, NHID)), full((NHID, NCLASS)), full((1, NCLASS)),
        ],
        out_specs=[
            pl.BlockSpec((BB, N, NCLASS), lambda b: (b, 0, 0)),
            pl.BlockSpec((BB, N, NCLASS), lambda b: (b, 0, 0)),
        ],
        out_shape=[
            jax.ShapeDtypeStruct((B, N, NCLASS), BF),
            jax.ShapeDtypeStruct((B, N, NCLASS), BF),
        ],
        compiler_params=pltpu.CompilerParams(dimension_semantics=("parallel",)),
    )(xs1, adjs1, xs2, adjs2, W1a, b1a2, W2a, b2a2, W1b, b1b2, W2b, b2b2)

    nk = N // NBLK

    out = pl.pallas_call(
        _proj_fuse_kernel,
        grid=(nk,),
        in_specs=[
            pl.BlockSpec((B, NBLK, NCLASS), lambda k: (0, k, 0)),
            pl.BlockSpec((NBLK, NCLASS, HID), lambda k: (k, 0, 0)),
            pl.BlockSpec((B, NBLK, NCLASS), lambda k: (0, k, 0)),
            pl.BlockSpec((NBLK, NCLASS, HID), lambda k: (k, 0, 0)),
            pl.BlockSpec((1, HID), lambda k: (0, 0)),
            pl.BlockSpec((1, HID), lambda k: (0, 0)),
            pl.BlockSpec((2 * HID, HID), lambda k: (0, 0)),
            pl.BlockSpec((1, HID), lambda k: (0, 0)),
        ],
        out_specs=pl.BlockSpec((B, HID), lambda k: (0, 0)),
        out_shape=jax.ShapeDtypeStruct((B, HID), jnp.float32),
        scratch_shapes=[
            pltpu.VMEM((B, HID), jnp.float32),
            pltpu.VMEM((B, HID), jnp.float32),
        ],
        compiler_params=pltpu.CompilerParams(dimension_semantics=("arbitrary",)),
    )(h2a, Pwa.reshape(N, NCLASS, HID), h2b, Pwb.reshape(N, NCLASS, HID),
      Pba.reshape(1, HID), Pbb.reshape(1, HID), Fw, Fb.reshape(1, HID))
    return out


# speedup vs baseline: 1.0959x; 1.0959x over previous
import jax
import jax.numpy as jnp
from jax.experimental import pallas as pl
from jax.experimental.pallas import tpu as pltpu

B, N, NFEAT, NHID, NCLASS, HID = 128, 256, 50, 512, 256, 256
NBLK = 16
BB = 8
BF = jnp.bfloat16


def _encoders_kernel(x1_ref, a1_ref, x2_ref, a2_ref,
                     w1a_ref, b1a_ref, w2a_ref, b2a_ref,
                     w1b_ref, b1b_ref, w2b_ref, b2b_ref,
                     h2a_ref, h2b_ref):
    w1a, w2a = w1a_ref[...].astype(BF), w2a_ref[...].astype(BF)
    w1b, w2b = w1b_ref[...].astype(BF), w2b_ref[...].astype(BF)
    b1a, b2a = b1a_ref[...], b2a_ref[...]
    b1b, b2b = b1b_ref[...], b2b_ref[...]

    def enc(x_ref, a_ref, w1, b1, w2, b2, out_ref):
        adjs = [a_ref[i].astype(BF) for i in range(BB)]
        ax = jnp.concatenate(
            [jnp.dot(adjs[i], jnp.transpose(x_ref[i]).astype(BF), preferred_element_type=jnp.float32).astype(BF)
             for i in range(BB)], axis=0)
        h1 = jnp.maximum(jnp.dot(ax, w1, preferred_element_type=jnp.float32) + b1, 0.0).astype(BF)
        g = jnp.dot(h1, w2, preferred_element_type=jnp.float32).astype(BF)
        for i in range(BB):
            out_ref[i] = (jnp.dot(adjs[i], g[i * N:(i + 1) * N], preferred_element_type=jnp.float32)
                          + b2).astype(BF)

    enc(x1_ref, a1_ref, w1a, b1a, w2a, b2a, h2a_ref)
    enc(x2_ref, a2_ref, w1b, b1b, w2b, b2b, h2b_ref)


def _proj_fuse_kernel(a1_ref, pw1_ref, a2_ref, pw2_ref,
                      pba_ref, pbb_ref, fw_ref, fb_ref,
                      out_ref, acc1, acc2):
    k = pl.program_id(0)

    @pl.when(k == 0)
    def _init():
        acc1[...] = jnp.zeros_like(acc1)
        acc2[...] = jnp.zeros_like(acc2)

    a1 = jnp.transpose(a1_ref[...], (1, 0, 2))
    a2 = jnp.transpose(a2_ref[...], (1, 0, 2))
    c1 = acc1[...]
    c2 = acc2[...]
    for j in range(NBLK):
        c1 += jnp.dot(a1[j], pw1_ref[j].astype(BF), preferred_element_type=jnp.float32)
        c2 += jnp.dot(a2[j], pw2_ref[j].astype(BF), preferred_element_type=jnp.float32)
    acc1[...] = c1
    acc2[...] = c2

    @pl.when(k == pl.num_programs(0) - 1)
    def _fin():
        e1 = acc1[...] + pba_ref[...]
        e2 = acc2[...] + pbb_ref[...]
        out_ref[...] = (jnp.dot(e1, fw_ref[0:HID, :], preferred_element_type=jnp.float32)
                        + jnp.dot(e2, fw_ref[HID:2 * HID, :], preferred_element_type=jnp.float32)
                        + fb_ref[...])


def kernel(xs1, adjs1, xs2, adjs2, W1a, b1a, W2a, b2a, Pwa, Pba,
           W1b, b1b, W2b, b2b, Pwb, Pbb, Fw, Fb):
    b1a2, b2a2 = b1a.reshape(1, NHID), b2a.reshape(1, NCLASS)
    b1b2, b2b2 = b1b.reshape(1, NHID), b2b.reshape(1, NCLASS)

    full = lambda shape: pl.BlockSpec(shape, lambda b: (0,) * len(shape))
    h2a, h2b = pl.pallas_call(
        _encoders_kernel,
        grid=(B // BB,),
        in_specs=[
            pl.BlockSpec((BB, NFEAT, N), lambda b: (b, 0, 0)),
            pl.BlockSpec((BB, N, N), lambda b: (b, 0, 0)),
            pl.BlockSpec((BB, NFEAT, N), lambda b: (b, 0, 0)),
            pl.BlockSpec((BB, N, N), lambda b: (b, 0, 0)),
            full((NFEAT, NHID)), full((1, NHID)), full((NHID, NCLASS)), full((1, NCLASS)),
            full((NFEAT, NHID)), full((1, NHID)), full((NHID, NCLASS)), full((1, NCLASS)),
        ],
        out_specs=[
            pl.BlockSpec((BB, N, NCLASS), lambda b: (b, 0, 0)),
            pl.BlockSpec((BB, N, NCLASS), lambda b: (b, 0, 0)),
        ],
        out_shape=[
            jax.ShapeDtypeStruct((B, N, NCLASS), BF),
            jax.ShapeDtypeStruct((B, N, NCLASS), BF),
        ],
        compiler_params=pltpu.CompilerParams(dimension_semantics=("parallel",)),
    )(xs1.swapaxes(1, 2), adjs1, xs2.swapaxes(1, 2), adjs2, W1a, b1a2, W2a, b2a2, W1b, b1b2, W2b, b2b2)

    nk = N // NBLK

    out = pl.pallas_call(
        _proj_fuse_kernel,
        grid=(nk,),
        in_specs=[
            pl.BlockSpec((B, NBLK, NCLASS), lambda k: (0, k, 0)),
            pl.BlockSpec((NBLK, NCLASS, HID), lambda k: (k, 0, 0)),
            pl.BlockSpec((B, NBLK, NCLASS), lambda k: (0, k, 0)),
            pl.BlockSpec((NBLK, NCLASS, HID), lambda k: (k, 0, 0)),
            pl.BlockSpec((1, HID), lambda k: (0, 0)),
            pl.BlockSpec((1, HID), lambda k: (0, 0)),
            pl.BlockSpec((2 * HID, HID), lambda k: (0, 0)),
            pl.BlockSpec((1, HID), lambda k: (0, 0)),
        ],
        out_specs=pl.BlockSpec((B, HID), lambda k: (0, 0)),
        out_shape=jax.ShapeDtypeStruct((B, HID), jnp.float32),
        scratch_shapes=[
            pltpu.VMEM((B, HID), jnp.float32),
            pltpu.VMEM((B, HID), jnp.float32),
        ],
        compiler_params=pltpu.CompilerParams(dimension_semantics=("arbitrary",)),
    )(h2a, Pwa.reshape(N, NCLASS, HID), h2b, Pwb.reshape(N, NCLASS, HID),
      Pba.reshape(1, HID), Pbb.reshape(1, HID), Fw, Fb.reshape(1, HID))
    return out


# speedup vs baseline: 1.1638x; 1.0619x over previous
import jax
import jax.numpy as jnp
from jax.experimental import pallas as pl
from jax.experimental.pallas import tpu as pltpu

B, N, NFEAT, NHID, NCLASS, HID = 128, 256, 50, 512, 256, 256
NBLK = 16
BB = 8
BF = jnp.bfloat16


def _encoders_kernel(x1_ref, a1_ref, x2_ref, a2_ref,
                     w1a_ref, b1a_ref, w2a_ref, b2a_ref,
                     w1b_ref, b1b_ref, w2b_ref, b2b_ref,
                     h2a_ref, h2b_ref):
    w1a, w2a = w1a_ref[...].astype(BF), w2a_ref[...].astype(BF)
    w1b, w2b = w1b_ref[...].astype(BF), w2b_ref[...].astype(BF)
    b1a, b2a = b1a_ref[...], b2a_ref[...]
    b1b, b2b = b1b_ref[...], b2b_ref[...]

    def enc(x_ref, a_ref, w1, b1, w2, b2, out_ref):
        adjs = [a_ref[i].astype(BF) for i in range(BB)]
        ax = jnp.concatenate(
            [jnp.dot(adjs[i], jnp.transpose(x_ref[i]), preferred_element_type=jnp.float32).astype(BF)
             for i in range(BB)], axis=0)
        h1 = jnp.maximum(jnp.dot(ax, w1, preferred_element_type=jnp.float32) + b1, 0.0).astype(BF)
        g = jnp.dot(h1, w2, preferred_element_type=jnp.float32).astype(BF)
        for i in range(BB):
            out_ref[i] = (jnp.dot(adjs[i], g[i * N:(i + 1) * N], preferred_element_type=jnp.float32)
                          + b2).astype(BF)

    enc(x1_ref, a1_ref, w1a, b1a, w2a, b2a, h2a_ref)
    enc(x2_ref, a2_ref, w1b, b1b, w2b, b2b, h2b_ref)


def _proj_fuse_kernel(a1_ref, pw1_ref, a2_ref, pw2_ref,
                      pba_ref, pbb_ref, fw_ref, fb_ref,
                      out_ref, acc1, acc2):
    k = pl.program_id(0)

    @pl.when(k == 0)
    def _init():
        acc1[...] = jnp.zeros_like(acc1)
        acc2[...] = jnp.zeros_like(acc2)

    a1 = jnp.transpose(a1_ref[...], (1, 0, 2))
    a2 = jnp.transpose(a2_ref[...], (1, 0, 2))
    c1 = acc1[...]
    c2 = acc2[...]
    for j in range(NBLK):
        c1 += jnp.dot(a1[j], pw1_ref[j].astype(BF), preferred_element_type=jnp.float32)
        c2 += jnp.dot(a2[j], pw2_ref[j].astype(BF), preferred_element_type=jnp.float32)
    acc1[...] = c1
    acc2[...] = c2

    @pl.when(k == pl.num_programs(0) - 1)
    def _fin():
        e1 = acc1[...] + pba_ref[...]
        e2 = acc2[...] + pbb_ref[...]
        out_ref[...] = (jnp.dot(e1, fw_ref[0:HID, :], preferred_element_type=jnp.float32)
                        + jnp.dot(e2, fw_ref[HID:2 * HID, :], preferred_element_type=jnp.float32)
                        + fb_ref[...])


def kernel(xs1, adjs1, xs2, adjs2, W1a, b1a, W2a, b2a, Pwa, Pba,
           W1b, b1b, W2b, b2b, Pwb, Pbb, Fw, Fb):
    b1a2, b2a2 = b1a.reshape(1, NHID), b2a.reshape(1, NCLASS)
    b1b2, b2b2 = b1b.reshape(1, NHID), b2b.reshape(1, NCLASS)

    full = lambda shape: pl.BlockSpec(shape, lambda b: (0,) * len(shape))
    h2a, h2b = pl.pallas_call(
        _encoders_kernel,
        grid=(B // BB,),
        in_specs=[
            pl.BlockSpec((BB, NFEAT, N), lambda b: (b, 0, 0)),
            pl.BlockSpec((BB, N, N), lambda b: (b, 0, 0)),
            pl.BlockSpec((BB, NFEAT, N), lambda b: (b, 0, 0)),
            pl.BlockSpec((BB, N, N), lambda b: (b, 0, 0)),
            full((NFEAT, NHID)), full((1, NHID)), full((NHID, NCLASS)), full((1, NCLASS)),
            full((NFEAT, NHID)), full((1, NHID)), full((NHID, NCLASS)), full((1, NCLASS)),
        ],
        out_specs=[
            pl.BlockSpec((BB, N, NCLASS), lambda b: (b, 0, 0)),
            pl.BlockSpec((BB, N, NCLASS), lambda b: (b, 0, 0)),
        ],
        out_shape=[
            jax.ShapeDtypeStruct((B, N, NCLASS), BF),
            jax.ShapeDtypeStruct((B, N, NCLASS), BF),
        ],
        compiler_params=pltpu.CompilerParams(dimension_semantics=("parallel",)),
    )(xs1.swapaxes(1, 2).astype(BF), adjs1, xs2.swapaxes(1, 2).astype(BF), adjs2, W1a, b1a2, W2a, b2a2, W1b, b1b2, W2b, b2b2)

    nk = N // NBLK

    out = pl.pallas_call(
        _proj_fuse_kernel,
        grid=(nk,),
        in_specs=[
            pl.BlockSpec((B, NBLK, NCLASS), lambda k: (0, k, 0)),
            pl.BlockSpec((NBLK, NCLASS, HID), lambda k: (k, 0, 0)),
            pl.BlockSpec((B, NBLK, NCLASS), lambda k: (0, k, 0)),
            pl.BlockSpec((NBLK, NCLASS, HID), lambda k: (k, 0, 0)),
            pl.BlockSpec((1, HID), lambda k: (0, 0)),
            pl.BlockSpec((1, HID), lambda k: (0, 0)),
            pl.BlockSpec((2 * HID, HID), lambda k: (0, 0)),
            pl.BlockSpec((1, HID), lambda k: (0, 0)),
        ],
        out_specs=pl.BlockSpec((B, HID), lambda k: (0, 0)),
        out_shape=jax.ShapeDtypeStruct((B, HID), jnp.float32),
        scratch_shapes=[
            pltpu.VMEM((B, HID), jnp.float32),
            pltpu.VMEM((B, HID), jnp.float32),
        ],
        compiler_params=pltpu.CompilerParams(dimension_semantics=("arbitrary",)),
    )(h2a, Pwa.reshape(N, NCLASS, HID), h2b, Pwb.reshape(N, NCLASS, HID),
      Pba.reshape(1, HID), Pbb.reshape(1, HID), Fw, Fb.reshape(1, HID))
    return out


# speedup vs baseline: 1.2024x; 1.0332x over previous
import jax
import jax.numpy as jnp
from jax.experimental import pallas as pl
from jax.experimental.pallas import tpu as pltpu

B, N, NFEAT, NHID, NCLASS, HID = 128, 256, 50, 512, 256, 256
NBLK = 16
BB = 16
BF = jnp.bfloat16


def _encoders_kernel(x1_ref, a1_ref, x2_ref, a2_ref,
                     w1a_ref, b1a_ref, w2a_ref, b2a_ref,
                     w1b_ref, b1b_ref, w2b_ref, b2b_ref,
                     h2a_ref, h2b_ref):
    w1a, w2a = w1a_ref[...].astype(BF), w2a_ref[...].astype(BF)
    w1b, w2b = w1b_ref[...].astype(BF), w2b_ref[...].astype(BF)
    b1a, b2a = b1a_ref[...], b2a_ref[...]
    b1b, b2b = b1b_ref[...], b2b_ref[...]

    def enc(x_ref, a_ref, w1, b1, w2, b2, out_ref):
        adjs = [a_ref[i].astype(BF) for i in range(BB)]
        ax = jnp.concatenate(
            [jnp.dot(adjs[i], jnp.transpose(x_ref[i]), preferred_element_type=jnp.float32).astype(BF)
             for i in range(BB)], axis=0)
        h1 = jnp.maximum(jnp.dot(ax, w1, preferred_element_type=jnp.float32) + b1, 0.0).astype(BF)
        g = jnp.dot(h1, w2, preferred_element_type=jnp.float32).astype(BF)
        for i in range(BB):
            out_ref[i] = (jnp.dot(adjs[i], g[i * N:(i + 1) * N], preferred_element_type=jnp.float32)
                          + b2).astype(BF)

    enc(x1_ref, a1_ref, w1a, b1a, w2a, b2a, h2a_ref)
    enc(x2_ref, a2_ref, w1b, b1b, w2b, b2b, h2b_ref)


def _proj_fuse_kernel(a1_ref, pw1_ref, a2_ref, pw2_ref,
                      pba_ref, pbb_ref, fw_ref, fb_ref,
                      out_ref, acc1, acc2):
    k = pl.program_id(0)

    @pl.when(k == 0)
    def _init():
        acc1[...] = jnp.zeros_like(acc1)
        acc2[...] = jnp.zeros_like(acc2)

    a1 = jnp.transpose(a1_ref[...], (1, 0, 2))
    a2 = jnp.transpose(a2_ref[...], (1, 0, 2))
    c1 = acc1[...]
    c2 = acc2[...]
    for j in range(NBLK):
        c1 += jnp.dot(a1[j], pw1_ref[j].astype(BF), preferred_element_type=jnp.float32)
        c2 += jnp.dot(a2[j], pw2_ref[j].astype(BF), preferred_element_type=jnp.float32)
    acc1[...] = c1
    acc2[...] = c2

    @pl.when(k == pl.num_programs(0) - 1)
    def _fin():
        e1 = acc1[...] + pba_ref[...]
        e2 = acc2[...] + pbb_ref[...]
        out_ref[...] = (jnp.dot(e1, fw_ref[0:HID, :], preferred_element_type=jnp.float32)
                        + jnp.dot(e2, fw_ref[HID:2 * HID, :], preferred_element_type=jnp.float32)
                        + fb_ref[...])


def kernel(xs1, adjs1, xs2, adjs2, W1a, b1a, W2a, b2a, Pwa, Pba,
           W1b, b1b, W2b, b2b, Pwb, Pbb, Fw, Fb):
    b1a2, b2a2 = b1a.reshape(1, NHID), b2a.reshape(1, NCLASS)
    b1b2, b2b2 = b1b.reshape(1, NHID), b2b.reshape(1, NCLASS)

    full = lambda shape: pl.BlockSpec(shape, lambda b: (0,) * len(shape))
    h2a, h2b = pl.pallas_call(
        _encoders_kernel,
        grid=(B // BB,),
        in_specs=[
            pl.BlockSpec((BB, NFEAT, N), lambda b: (b, 0, 0)),
            pl.BlockSpec((BB, N, N), lambda b: (b, 0, 0)),
            pl.BlockSpec((BB, NFEAT, N), lambda b: (b, 0, 0)),
            pl.BlockSpec((BB, N, N), lambda b: (b, 0, 0)),
            full((NFEAT, NHID)), full((1, NHID)), full((NHID, NCLASS)), full((1, NCLASS)),
            full((NFEAT, NHID)), full((1, NHID)), full((NHID, NCLASS)), full((1, NCLASS)),
        ],
        out_specs=[
            pl.BlockSpec((BB, N, NCLASS), lambda b: (b, 0, 0)),
            pl.BlockSpec((BB, N, NCLASS), lambda b: (b, 0, 0)),
        ],
        out_shape=[
            jax.ShapeDtypeStruct((B, N, NCLASS), BF),
            jax.ShapeDtypeStruct((B, N, NCLASS), BF),
        ],
        compiler_params=pltpu.CompilerParams(dimension_semantics=("parallel",)),
    )(xs1.swapaxes(1, 2).astype(BF), adjs1, xs2.swapaxes(1, 2).astype(BF), adjs2, W1a, b1a2, W2a, b2a2, W1b, b1b2, W2b, b2b2)

    nk = N // NBLK

    out = pl.pallas_call(
        _proj_fuse_kernel,
        grid=(nk,),
        in_specs=[
            pl.BlockSpec((B, NBLK, NCLASS), lambda k: (0, k, 0)),
            pl.BlockSpec((NBLK, NCLASS, HID), lambda k: (k, 0, 0)),
            pl.BlockSpec((B, NBLK, NCLASS), lambda k: (0, k, 0)),
            pl.BlockSpec((NBLK, NCLASS, HID), lambda k: (k, 0, 0)),
            pl.BlockSpec((1, HID), lambda k: (0, 0)),
            pl.BlockSpec((1, HID), lambda k: (0, 0)),
            pl.BlockSpec((2 * HID, HID), lambda k: (0, 0)),
            pl.BlockSpec((1, HID), lambda k: (0, 0)),
        ],
        out_specs=pl.BlockSpec((B, HID), lambda k: (0, 0)),
        out_shape=jax.ShapeDtypeStruct((B, HID), jnp.float32),
        scratch_shapes=[
            pltpu.VMEM((B, HID), jnp.float32),
            pltpu.VMEM((B, HID), jnp.float32),
        ],
        compiler_params=pltpu.CompilerParams(dimension_semantics=("arbitrary",)),
    )(h2a, Pwa.reshape(N, NCLASS, HID), h2b, Pwb.reshape(N, NCLASS, HID),
      Pba.reshape(1, HID), Pbb.reshape(1, HID), Fw, Fb.reshape(1, HID))
    return out
